# initial kernel scaffold (unmeasured)
import jax
import jax.numpy as jnp
from jax import lax
from jax.experimental import pallas as pl
from jax.experimental.pallas import tpu as pltpu

N_DEV = 4
M, K_LOC, N = 4096, 1024, 8192
M_PER = M // N_DEV
NH = N // 2
F32 = jnp.float32


def _gemm(x, w):

    def body(x_ref, w_ref, o_ref):
        o_ref[...] = jnp.dot(
            x_ref[...], w_ref[...], preferred_element_type=F32
        )

    return pl.pallas_call(
        body,
        grid=(4, 4),
        in_specs=[
            pl.BlockSpec((M_PER, K_LOC), lambda i, j: (i, 0)),
            pl.BlockSpec((K_LOC, 2048), lambda i, j: (0, j)),
        ],
        out_specs=pl.BlockSpec((M_PER, 2048), lambda i, j: (i, j)),
        out_shape=jax.ShapeDtypeStruct((M, N), F32),
    )(x, w)


def _rs_epilogue(partial):

    def body(
        p_ref,
        o_ref,
        rbuf,
        acc,
        tmp,
        amax_mine,
        amax_rbuf,
        ssem,
        rsem,
        a_ssem,
        a_rsem,
        csem,
    ):
        my = lax.axis_index("i")
        right = lax.rem(my + 1, N_DEV)
        left = lax.rem(my + 3, N_DEV)

        barrier = pltpu.get_barrier_semaphore()
        for nbr in (left, right):
            pl.semaphore_signal(
                barrier, inc=1, device_id=(nbr,),
                device_id_type=pl.DeviceIdType.MESH,
            )
        pl.semaphore_wait(barrier, 2)

        def send_chunk_row(dirn, s):
            if dirn == 0:
                return lax.rem(my + 3 - s + N_DEV, N_DEV)
            return lax.rem(my + 1 + s, N_DEV)

        def recv_chunk_row(dirn, s):
            if dirn == 0:
                return lax.rem(my + 2 - s + N_DEV, N_DEV)
            return lax.rem(my + 2 + s, N_DEV)

        mloc = None
        for s in range(3):
            rdmas = []
            for d in range(2):
                if s == 0:
                    c = send_chunk_row(d, 0)
                    src = p_ref.at[
                        pl.ds(c * M_PER, M_PER), pl.ds(d * NH, NH)
                    ]
                else:
                    src = acc.at[d]
                tgt = right if d == 0 else left
                r = pltpu.make_async_remote_copy(
                    src_ref=src,
                    dst_ref=rbuf.at[d, s],
                    send_sem=ssem.at[d, s],
                    recv_sem=rsem.at[d, s],
                    device_id=(tgt,),
                    device_id_type=pl.DeviceIdType.MESH,
                )
                r.start()
                rdmas.append(r)
            for r in rdmas:
                r.wait()

            for d in range(2):
                cp_r = pltpu.make_async_copy(rbuf.at[d, s], acc.at[d], csem.at[0])
                cp_r.start()
                c = recv_chunk_row(d, s)
                cp_p = pltpu.make_async_copy(
                    p_ref.at[pl.ds(c * M_PER, M_PER), pl.ds(d * NH, NH)],
                    tmp,
                    csem.at[1],
                )
                cp_p.start()
                cp_r.wait()
                cp_p.wait()
                if s < 2:
                    acc[d] = acc[d] + tmp[...]
                else:
                    res = jnp.maximum(acc[d] + tmp[...], 0.0)
                    acc[d] = res
                    m = jnp.max(res)
                    mloc = m if mloc is None else jnp.maximum(mloc, m)

        amax_mine[...] = jnp.full((8, 128), mloc, F32)
        ardmas = []
        for o in (1, 2, 3):
            tgt = lax.rem(my + o, N_DEV)
            r = pltpu.make_async_remote_copy(
                src_ref=amax_mine,
                dst_ref=amax_rbuf.at[3 - o],
                send_sem=a_ssem.at[o - 1],
                recv_sem=a_rsem.at[3 - o],
                device_id=(tgt,),
                device_id_type=pl.DeviceIdType.MESH,
            )
            r.start()
            ardmas.append(r)
        for r in ardmas:
            r.wait()
        gmax = jnp.maximum(mloc, jnp.max(amax_rbuf[...]))
        scale = gmax / 127.0

        for d in range(2):
            tmp[...] = jnp.clip(jnp.round(acc[d] / scale), 0.0, 127.0) * scale
            cp = pltpu.make_async_copy(
                tmp, o_ref.at[:, pl.ds(d * NH, NH)], csem.at[0]
            )
            cp.start()
            cp.wait()

    return pl.pallas_call(
        body,
        out_shape=jax.ShapeDtypeStruct((M_PER, N), F32),
        in_specs=[pl.BlockSpec(memory_space=pl.ANY)],
        out_specs=pl.BlockSpec(memory_space=pl.ANY),
        scratch_shapes=[
            pltpu.HBM((2, 3, M_PER, NH), F32),
            pltpu.VMEM((2, M_PER, NH), F32),
            pltpu.VMEM((M_PER, NH), F32),
            pltpu.VMEM((8, 128), F32),
            pltpu.VMEM((3, 8, 128), F32),
            pltpu.SemaphoreType.DMA((2, 3)),
            pltpu.SemaphoreType.DMA((2, 3)),
            pltpu.SemaphoreType.DMA((3,)),
            pltpu.SemaphoreType.DMA((3,)),
            pltpu.SemaphoreType.DMA((2,)),
        ],
        compiler_params=pltpu.CompilerParams(collective_id=0),
    )(partial)


def kernel(x, w_mat):
    partial = _gemm(x, w_mat)
    return _rs_epilogue(partial)


# baseline (device time: 819284 ns/iter reference)
import jax
import jax.numpy as jnp
from jax import lax
from jax.experimental import pallas as pl
from jax.experimental.pallas import tpu as pltpu

N_DEV = 4
M, K_LOC, N = 4096, 1024, 8192
M_PER = M // N_DEV
NH = N // 2
F32 = jnp.float32


def _gemm(x, w):

    def body(x_ref, w_ref, o_ref):
        o_ref[...] = jnp.dot(
            x_ref[...], w_ref[...], preferred_element_type=F32
        )

    return pl.pallas_call(
        body,
        grid=(4, 4),
        in_specs=[
            pl.BlockSpec((M_PER, K_LOC), lambda i, j: (i, 0)),
            pl.BlockSpec((K_LOC, 2048), lambda i, j: (0, j)),
        ],
        out_specs=pl.BlockSpec((M_PER, 2048), lambda i, j: (i, j)),
        out_shape=jax.ShapeDtypeStruct((M, N), F32),
    )(x, w)


def _rs_epilogue(partial):

    def body(
        p_ref,
        o_ref,
        rbuf,
        sbuf,
        va,
        vb,
        amax_mine,
        amax_rbuf,
        ssem,
        rsem,
        a_ssem,
        a_rsem,
        csem,
    ):
        my = lax.axis_index("i")
        right = lax.rem(my + 1, N_DEV)
        left = lax.rem(my + 3, N_DEV)

        barrier = pltpu.get_barrier_semaphore()
        for nbr in (left, right):
            pl.semaphore_signal(
                barrier, inc=1, device_id=(nbr,),
                device_id_type=pl.DeviceIdType.MESH,
            )
        pl.semaphore_wait(barrier, 2)

        def send_chunk_row(dirn, s):
            if dirn == 0:
                return lax.rem(my + 3 - s + N_DEV, N_DEV)
            return lax.rem(my + 1 + s, N_DEV)

        def recv_chunk_row(dirn, s):
            if dirn == 0:
                return lax.rem(my + 2 - s + N_DEV, N_DEV)
            return lax.rem(my + 2 + s, N_DEV)

        mloc = None
        for s in range(3):
            rdmas = []
            for d in range(2):
                if s == 0:
                    c = send_chunk_row(d, 0)
                    src = p_ref.at[
                        pl.ds(c * M_PER, M_PER), pl.ds(d * NH, NH)
                    ]
                else:
                    src = sbuf.at[d]
                tgt = right if d == 0 else left
                r = pltpu.make_async_remote_copy(
                    src_ref=src,
                    dst_ref=rbuf.at[d, s],
                    send_sem=ssem.at[d, s],
                    recv_sem=rsem.at[d, s],
                    device_id=(tgt,),
                    device_id_type=pl.DeviceIdType.MESH,
                )
                r.start()
                rdmas.append(r)
            for r in rdmas:
                r.wait()

            for d in range(2):
                cp_r = pltpu.make_async_copy(rbuf.at[d, s], va, csem.at[0])
                cp_r.start()
                c = recv_chunk_row(d, s)
                cp_p = pltpu.make_async_copy(
                    p_ref.at[pl.ds(c * M_PER, M_PER), pl.ds(d * NH, NH)],
                    vb,
                    csem.at[1],
                )
                cp_p.start()
                cp_r.wait()
                cp_p.wait()
                if s < 2:
                    va[...] = va[...] + vb[...]
                else:
                    res = jnp.maximum(va[...] + vb[...], 0.0)
                    va[...] = res
                    m = jnp.max(res)
                    mloc = m if mloc is None else jnp.maximum(mloc, m)
                cp_o = pltpu.make_async_copy(va, sbuf.at[d], csem.at[0])
                cp_o.start()
                cp_o.wait()

        amax_mine[...] = jnp.full((8, 128), mloc, F32)
        ardmas = []
        for o in (1, 2, 3):
            tgt = lax.rem(my + o, N_DEV)
            r = pltpu.make_async_remote_copy(
                src_ref=amax_mine,
                dst_ref=amax_rbuf.at[3 - o],
                send_sem=a_ssem.at[o - 1],
                recv_sem=a_rsem.at[3 - o],
                device_id=(tgt,),
                device_id_type=pl.DeviceIdType.MESH,
            )
            r.start()
            ardmas.append(r)
        for r in ardmas:
            r.wait()
        gmax = jnp.maximum(mloc, jnp.max(amax_rbuf[...]))
        scale = gmax / 127.0

        for d in range(2):
            cp_i = pltpu.make_async_copy(sbuf.at[d], va, csem.at[0])
            cp_i.start()
            cp_i.wait()
            va[...] = jnp.clip(jnp.round(va[...] / scale), 0.0, 127.0) * scale
            cp = pltpu.make_async_copy(
                va, o_ref.at[:, pl.ds(d * NH, NH)], csem.at[0]
            )
            cp.start()
            cp.wait()

    return pl.pallas_call(
        body,
        out_shape=(
            jax.ShapeDtypeStruct((M_PER, N), F32),
            jax.ShapeDtypeStruct((2, 3, M_PER, NH), F32),
            jax.ShapeDtypeStruct((2, M_PER, NH), F32),
        ),
        in_specs=[pl.BlockSpec(memory_space=pl.ANY)],
        out_specs=(
            pl.BlockSpec(memory_space=pl.ANY),
            pl.BlockSpec(memory_space=pl.ANY),
            pl.BlockSpec(memory_space=pl.ANY),
        ),
        scratch_shapes=[
            pltpu.VMEM((M_PER, NH), F32),
            pltpu.VMEM((M_PER, NH), F32),
            pltpu.VMEM((8, 128), F32),
            pltpu.VMEM((3, 8, 128), F32),
            pltpu.SemaphoreType.DMA((2, 3)),
            pltpu.SemaphoreType.DMA((2, 3)),
            pltpu.SemaphoreType.DMA((3,)),
            pltpu.SemaphoreType.DMA((3,)),
            pltpu.SemaphoreType.DMA((2,)),
        ],
        compiler_params=pltpu.CompilerParams(
            collective_id=0, vmem_limit_bytes=60 * 1024 * 1024
        ),
    )(partial)[0]


def kernel(x, w_mat):
    partial = _gemm(x, w_mat)
    return _rs_epilogue(partial)


# device time: 634400 ns/iter; 1.2914x vs baseline; 1.2914x over previous
import jax
import jax.numpy as jnp
from jax import lax
from jax.experimental import pallas as pl
from jax.experimental.pallas import tpu as pltpu

N_DEV = 4
M, K_LOC, N = 4096, 1024, 8192
M_PER = M // N_DEV
NH = N // 2
HT = NH // 2
F32 = jnp.float32


def kernel(x, w_mat):
    def body(
        x_ref,
        w_ref,
        o_ref,
        rbuf,
        sbuf,
        g,
        xv,
        wv,
        amax_mine,
        amax_rbuf,
        ssem,
        rsem,
        a_ssem,
        a_rsem,
        csem,
    ):
        my = lax.axis_index("i")
        right = lax.rem(my + 1, N_DEV)
        left = lax.rem(my + 3, N_DEV)

        barrier = pltpu.get_barrier_semaphore()
        for nbr in (left, right):
            pl.semaphore_signal(
                barrier, inc=1, device_id=(nbr,),
                device_id_type=pl.DeviceIdType.MESH,
            )
        pl.semaphore_wait(barrier, 2)

        def sc(d, s):
            if d == 0:
                return lax.rem(my + 3 - s + N_DEV, N_DEV)
            return lax.rem(my + 1 + s, N_DEV)

        def rc(d, s):
            if d == 0:
                return lax.rem(my + 2 - s + N_DEV, N_DEV)
            return lax.rem(my + 2 + s, N_DEV)

        def load_x(c):
            cp = pltpu.make_async_copy(
                x_ref.at[pl.ds(c * M_PER, M_PER), :], xv, csem.at[0]
            )
            cp.start()
            cp.wait()

        def load_w(d, h):
            cp = pltpu.make_async_copy(
                w_ref.at[:, pl.ds(d * NH + h * HT, HT)], wv, csem.at[0]
            )
            cp.start()
            cp.wait()

        def mk_send(d, s, h):
            return pltpu.make_async_remote_copy(
                src_ref=sbuf.at[d, :, pl.ds(h * HT, HT)],
                dst_ref=rbuf.at[d, s, :, pl.ds(h * HT, HT)],
                send_sem=ssem.at[d, s, h],
                recv_sem=rsem.at[d, s, h],
                device_id=(right if d == 0 else left,),
                device_id_type=pl.DeviceIdType.MESH,
            )

        sends = {}
        for d in (0, 1):
            load_x(sc(d, 0))
            for h in (0, 1):
                load_w(d, h)
                g[d, :, pl.ds(h * HT, HT)] = jnp.dot(
                    xv[...], wv[...], preferred_element_type=F32
                )
                cp = pltpu.make_async_copy(
                    g.at[d, :, pl.ds(h * HT, HT)],
                    sbuf.at[d, :, pl.ds(h * HT, HT)],
                    csem.at[1],
                )
                cp.start()
                cp.wait()
                sends[(0, d, h)] = mk_send(d, 0, h)
                sends[(0, d, h)].start()

        for d in (0, 1):
            load_x(rc(d, 0))
            for h in (0, 1):
                load_w(d, h)
                g[d, :, pl.ds(h * HT, HT)] = jnp.dot(
                    xv[...], wv[...], preferred_element_type=F32
                )

        mvals = []
        for s in range(3):
            for d, h in ((0, 0), (1, 0), (0, 1), (1, 1)):
                sd = sends[(s, d, h)]
                sd.wait_recv()
                cp = pltpu.make_async_copy(
                    rbuf.at[d, s, :, pl.ds(h * HT, HT)], wv, csem.at[0]
                )
                cp.start()
                cp.wait()
                if s < 2:
                    wv[...] = wv[...] + g[d, :, pl.ds(h * HT, HT)]
                    sd.wait_send()
                    cp2 = pltpu.make_async_copy(
                        wv, sbuf.at[d, :, pl.ds(h * HT, HT)], csem.at[1]
                    )
                    cp2.start()
                    cp2.wait()
                    sends[(s + 1, d, h)] = mk_send(d, s + 1, h)
                    sends[(s + 1, d, h)].start()
                else:
                    res = jnp.maximum(wv[...] + g[d, :, pl.ds(h * HT, HT)], 0.0)
                    g[d, :, pl.ds(h * HT, HT)] = res
                    mvals.append(jnp.max(res))
            if s < 2:
                for d in (0, 1):
                    load_x(rc(d, s + 1))
                    for h in (0, 1):
                        load_w(d, h)
                        g[d, :, pl.ds(h * HT, HT)] = jnp.dot(
                            xv[...], wv[...], preferred_element_type=F32
                        )

        for d, h in ((0, 0), (1, 0), (0, 1), (1, 1)):
            sends[(2, d, h)].wait_send()

        mloc = mvals[0]
        for m in mvals[1:]:
            mloc = jnp.maximum(mloc, m)
        amax_mine[...] = jnp.full((8, 128), mloc, F32)
        ardmas = []
        for o in (1, 2, 3):
            r = pltpu.make_async_remote_copy(
                src_ref=amax_mine,
                dst_ref=amax_rbuf.at[3 - o],
                send_sem=a_ssem.at[o - 1],
                recv_sem=a_rsem.at[3 - o],
                device_id=(lax.rem(my + o, N_DEV),),
                device_id_type=pl.DeviceIdType.MESH,
            )
            r.start()
            ardmas.append(r)
        for r in ardmas:
            r.wait()
        gmax = jnp.maximum(mloc, jnp.max(amax_rbuf[...]))
        scale = gmax / 127.0

        for d in (0, 1):
            for h in (0, 1):
                wv[...] = (
                    jnp.clip(
                        jnp.round(g[d, :, pl.ds(h * HT, HT)] / scale),
                        0.0,
                        127.0,
                    )
                    * scale
                )
                cp = pltpu.make_async_copy(
                    wv, o_ref.at[:, pl.ds(d * NH + h * HT, HT)], csem.at[0]
                )
                cp.start()
                cp.wait()

    outs = pl.pallas_call(
        body,
        out_shape=(
            jax.ShapeDtypeStruct((M_PER, N), F32),
            jax.ShapeDtypeStruct((2, 3, M_PER, NH), F32),
            jax.ShapeDtypeStruct((2, M_PER, NH), F32),
        ),
        in_specs=[
            pl.BlockSpec(memory_space=pl.ANY),
            pl.BlockSpec(memory_space=pl.ANY),
        ],
        out_specs=(
            pl.BlockSpec(memory_space=pl.ANY),
            pl.BlockSpec(memory_space=pl.ANY),
            pl.BlockSpec(memory_space=pl.ANY),
        ),
        scratch_shapes=[
            pltpu.VMEM((2, M_PER, NH), F32),
            pltpu.VMEM((M_PER, K_LOC), F32),
            pltpu.VMEM((K_LOC, HT), F32),
            pltpu.VMEM((8, 128), F32),
            pltpu.VMEM((3, 8, 128), F32),
            pltpu.SemaphoreType.DMA((2, 3, 2)),
            pltpu.SemaphoreType.DMA((2, 3, 2)),
            pltpu.SemaphoreType.DMA((3,)),
            pltpu.SemaphoreType.DMA((3,)),
            pltpu.SemaphoreType.DMA((2,)),
        ],
        compiler_params=pltpu.CompilerParams(
            collective_id=0, vmem_limit_bytes=58 * 1024 * 1024
        ),
    )(x, w_mat)
    return outs[0]


# device time: 347844 ns/iter; 2.3553x vs baseline; 1.8238x over previous
import jax
import jax.numpy as jnp
from jax import lax
from jax.experimental import pallas as pl
from jax.experimental.pallas import tpu as pltpu

N_DEV = 4
M, K_LOC, N = 4096, 1024, 8192
M_PER = M // N_DEV
NH = N // 2
HT = NH // 2
F32 = jnp.float32
BF16 = jnp.bfloat16


def kernel(x, w_mat):
    def body(
        x_ref,
        w_ref,
        o_ref,
        rbuf,
        sbuf,
        g,
        xv,
        wv,
        wvb,
        amax_mine,
        amax_rbuf,
        ssem,
        rsem,
        a_ssem,
        a_rsem,
        csem,
    ):
        my = lax.axis_index("i")
        right = lax.rem(my + 1, N_DEV)
        left = lax.rem(my + 3, N_DEV)

        barrier = pltpu.get_barrier_semaphore()
        for nbr in (left, right):
            pl.semaphore_signal(
                barrier, inc=1, device_id=(nbr,),
                device_id_type=pl.DeviceIdType.MESH,
            )
        pl.semaphore_wait(barrier, 2)

        def sc(d, s):
            return jnp.where(
                d == 0, lax.rem(my + 3 - s + N_DEV, N_DEV),
                lax.rem(my + 1 + s, N_DEV),
            )

        def rc(d, s):
            return jnp.where(
                d == 0, lax.rem(my + 2 - s + N_DEV, N_DEV),
                lax.rem(my + 2 + s, N_DEV),
            )

        def load_x(c):
            cp = pltpu.make_async_copy(
                x_ref.at[pl.ds(c * M_PER, M_PER), :], xv, csem.at[0]
            )
            cp.start()
            cp.wait()

        def load_w(d, h):
            cp = pltpu.make_async_copy(
                w_ref.at[:, pl.ds(d * NH + h * HT, HT)], wv, csem.at[0]
            )
            cp.start()
            cp.wait()

        def mk_send(d, s, h):
            return pltpu.make_async_remote_copy(
                src_ref=sbuf.at[d, :, pl.ds(h * HT, HT)],
                dst_ref=rbuf.at[d, s, :, pl.ds(h * HT, HT)],
                send_sem=ssem.at[d, s, h],
                recv_sem=rsem.at[d, s, h],
                device_id=(jnp.where(d == 0, right, left),),
                device_id_type=pl.DeviceIdType.MESH,
            )

        def start_i(i, carry):
            d = lax.rem(i, 2)
            h = i // 2
            load_x(sc(d, 0))
            load_w(d, h)
            wvb[...] = jnp.dot(
                xv[...], wv[...], preferred_element_type=F32
            ).astype(BF16)
            cp = pltpu.make_async_copy(
                wvb, sbuf.at[d, :, pl.ds(h * HT, HT)], csem.at[1]
            )
            cp.start()
            cp.wait()
            mk_send(d, 0, h).start()
            return carry

        lax.fori_loop(0, 4, start_i, 0)

        def precompute(s):
            for d in (0, 1):
                load_x(rc(d, s))

                def pre_h(h, carry, d=d):
                    load_w(d, h)
                    g[d, :, pl.ds(h * HT, HT)] = jnp.dot(
                        xv[...], wv[...], preferred_element_type=F32
                    )
                    return carry

                lax.fori_loop(0, 2, pre_h, 0)

        precompute(0)

        def hop(s, carry):
            def half_i(i, c2):
                d = lax.rem(i, 2)
                h = i // 2
                sd = mk_send(d, s, h)
                sd.wait_recv()
                cp = pltpu.make_async_copy(
                    rbuf.at[d, s, :, pl.ds(h * HT, HT)], wvb, csem.at[0]
                )
                cp.start()
                cp.wait()
                acc = wvb[...].astype(F32) + g[d, :, pl.ds(h * HT, HT)]
                wvb[...] = acc.astype(BF16)
                sd.wait_send()
                cp2 = pltpu.make_async_copy(
                    wvb, sbuf.at[d, :, pl.ds(h * HT, HT)], csem.at[1]
                )
                cp2.start()
                cp2.wait()
                mk_send(d, s + 1, h).start()
                return c2

            lax.fori_loop(0, 4, half_i, 0)
            precompute(s + 1)
            return carry

        lax.fori_loop(0, 2, hop, 0)

        def fin_i(i, m):
            d = lax.rem(i, 2)
            h = i // 2
            mk_send(d, 2, h).wait_recv()
            cp = pltpu.make_async_copy(
                rbuf.at[d, 2, :, pl.ds(h * HT, HT)], wvb, csem.at[0]
            )
            cp.start()
            cp.wait()
            res = jnp.maximum(
                wvb[...].astype(F32) + g[d, :, pl.ds(h * HT, HT)], 0.0
            )
            g[d, :, pl.ds(h * HT, HT)] = res
            return jnp.maximum(m, jnp.max(res))

        mloc = lax.fori_loop(0, 4, fin_i, jnp.float32(0.0))

        def ws_i(i, carry):
            mk_send(lax.rem(i, 2), 2, i // 2).wait_send()
            return carry

        lax.fori_loop(0, 4, ws_i, 0)

        amax_mine[...] = jnp.full((8, 128), mloc, F32)
        ardmas = []
        for o in (1, 2, 3):
            r = pltpu.make_async_remote_copy(
                src_ref=amax_mine,
                dst_ref=amax_rbuf.at[3 - o],
                send_sem=a_ssem.at[o - 1],
                recv_sem=a_rsem.at[3 - o],
                device_id=(lax.rem(my + o, N_DEV),),
                device_id_type=pl.DeviceIdType.MESH,
            )
            r.start()
            ardmas.append(r)
        for r in ardmas:
            r.wait()
        gmax = jnp.maximum(mloc, jnp.max(amax_rbuf[...]))
        scale = gmax / 127.0

        def q_i(i, carry):
            d = lax.rem(i, 2)
            h = i // 2
            wv[...] = (
                jnp.clip(
                    jnp.round(g[d, :, pl.ds(h * HT, HT)] / scale), 0.0, 127.0
                )
                * scale
            )
            cp = pltpu.make_async_copy(
                wv, o_ref.at[:, pl.ds(d * NH + h * HT, HT)], csem.at[0]
            )
            cp.start()
            cp.wait()
            return carry

        lax.fori_loop(0, 4, q_i, 0)

    outs = pl.pallas_call(
        body,
        out_shape=(
            jax.ShapeDtypeStruct((M_PER, N), F32),
            jax.ShapeDtypeStruct((2, 3, M_PER, NH), BF16),
            jax.ShapeDtypeStruct((2, M_PER, NH), BF16),
        ),
        in_specs=[
            pl.BlockSpec(memory_space=pl.ANY),
            pl.BlockSpec(memory_space=pl.ANY),
        ],
        out_specs=(
            pl.BlockSpec(memory_space=pl.ANY),
            pl.BlockSpec(memory_space=pl.ANY),
            pl.BlockSpec(memory_space=pl.ANY),
        ),
        scratch_shapes=[
            pltpu.VMEM((2, M_PER, NH), F32),
            pltpu.VMEM((M_PER, K_LOC), F32),
            pltpu.VMEM((K_LOC, HT), F32),
            pltpu.VMEM((M_PER, HT), BF16),
            pltpu.VMEM((8, 128), F32),
            pltpu.VMEM((3, 8, 128), F32),
            pltpu.SemaphoreType.DMA((2, 3, 2)),
            pltpu.SemaphoreType.DMA((2, 3, 2)),
            pltpu.SemaphoreType.DMA((3,)),
            pltpu.SemaphoreType.DMA((3,)),
            pltpu.SemaphoreType.DMA((2,)),
        ],
        compiler_params=pltpu.CompilerParams(
            collective_id=0, vmem_limit_bytes=58 * 1024 * 1024
        ),
    )(x, w_mat)
    return outs[0]


# device time: 346131 ns/iter; 2.3670x vs baseline; 1.0049x over previous
import jax
import jax.numpy as jnp
from jax import lax
from jax.experimental import pallas as pl
from jax.experimental.pallas import tpu as pltpu

N_DEV = 4
M, K_LOC, N = 4096, 1024, 8192
M_PER = M // N_DEV
NH = N // 2
HT = NH // 2
F32 = jnp.float32
BF16 = jnp.bfloat16


def kernel(x, w_mat):
    def body(
        x_ref,
        w_ref,
        o_ref,
        rbuf,
        sbuf,
        g,
        xv,
        wv,
        wvb,
        amax_mine,
        amax_rbuf,
        ssem,
        rsem,
        a_ssem,
        a_rsem,
        csem,
    ):
        my = lax.axis_index("i")
        right = lax.rem(my + 1, N_DEV)
        left = lax.rem(my + 3, N_DEV)

        barrier = pltpu.get_barrier_semaphore()
        for nbr in (left, right):
            pl.semaphore_signal(
                barrier, inc=1, device_id=(nbr,),
                device_id_type=pl.DeviceIdType.MESH,
            )
        pl.semaphore_wait(barrier, 2)

        def sc(d, s):
            return jnp.where(
                d == 0, lax.rem(my + 3 - s + N_DEV, N_DEV),
                lax.rem(my + 1 + s, N_DEV),
            )

        def rc(d, s):
            return jnp.where(
                d == 0, lax.rem(my + 2 - s + N_DEV, N_DEV),
                lax.rem(my + 2 + s, N_DEV),
            )

        def load_x(c):
            cp = pltpu.make_async_copy(
                x_ref.at[pl.ds(c * M_PER, M_PER), :], xv, csem.at[0]
            )
            cp.start()
            cp.wait()

        def load_w(d, h):
            cp = pltpu.make_async_copy(
                w_ref.at[:, pl.ds(d * NH + h * HT, HT)], wv, csem.at[0]
            )
            cp.start()
            cp.wait()

        def mk_send(d, s, h):
            return pltpu.make_async_remote_copy(
                src_ref=sbuf.at[d, :, pl.ds(h * HT, HT)],
                dst_ref=rbuf.at[d, s, :, pl.ds(h * HT, HT)],
                send_sem=ssem.at[d, s, h],
                recv_sem=rsem.at[d, s, h],
                device_id=(jnp.where(d == 0, right, left),),
                device_id_type=pl.DeviceIdType.MESH,
            )

        def start_i(i, carry):
            d = lax.rem(i, 2)
            h = i // 2
            load_x(sc(d, 0))
            load_w(d, h)
            wvb[...] = jnp.dot(
                xv[...], wv[...], preferred_element_type=F32
            ).astype(BF16)
            cp = pltpu.make_async_copy(
                wvb, sbuf.at[d, :, pl.ds(h * HT, HT)], csem.at[1]
            )
            cp.start()
            cp.wait()
            mk_send(d, 0, h).start()
            return carry

        lax.fori_loop(0, 4, start_i, 0)

        def precompute(s):
            for d in (0, 1):
                load_x(rc(d, s))

                def pre_h(h, carry, d=d):
                    load_w(d, h)
                    g[d, :, pl.ds(h * HT, HT)] = jnp.dot(
                        xv[...], wv[...], preferred_element_type=F32
                    )
                    return carry

                lax.fori_loop(0, 2, pre_h, 0)

        precompute(0)

        def hop(s, carry):
            def half_i(i, c2):
                d = lax.rem(i, 2)
                h = i // 2
                sd = mk_send(d, s, h)
                sd.wait_recv()
                cp = pltpu.make_async_copy(
                    rbuf.at[d, s, :, pl.ds(h * HT, HT)], wvb, csem.at[0]
                )
                cp.start()
                cp.wait()
                acc = wvb[...].astype(F32) + g[d, :, pl.ds(h * HT, HT)]
                wvb[...] = acc.astype(BF16)
                sd.wait_send()
                cp2 = pltpu.make_async_copy(
                    wvb, sbuf.at[d, :, pl.ds(h * HT, HT)], csem.at[1]
                )
                cp2.start()
                cp2.wait()
                mk_send(d, s + 1, h).start()
                return c2

            lax.fori_loop(0, 4, half_i, 0)
            precompute(s + 1)
            return carry

        lax.fori_loop(0, 2, hop, 0)

        def fin_i(i, m):
            d = lax.rem(i, 2)
            h = i // 2
            mk_send(d, 2, h).wait_recv()
            cp = pltpu.make_async_copy(
                rbuf.at[d, 2, :, pl.ds(h * HT, HT)], wvb, csem.at[0]
            )
            cp.start()
            cp.wait()
            res = jnp.maximum(
                wvb[...].astype(F32) + g[d, :, pl.ds(h * HT, HT)], 0.0
            )
            g[d, :, pl.ds(h * HT, HT)] = res
            return jnp.maximum(m, jnp.max(res))

        mloc = lax.fori_loop(0, 4, fin_i, jnp.float32(0.0))

        def ws_i(i, carry):
            mk_send(lax.rem(i, 2), 2, i // 2).wait_send()
            return carry

        lax.fori_loop(0, 4, ws_i, 0)

        amax_mine[...] = jnp.full((8, 128), mloc, F32)
        ardmas = []
        for o in (1, 2, 3):
            r = pltpu.make_async_remote_copy(
                src_ref=amax_mine,
                dst_ref=amax_rbuf.at[3 - o],
                send_sem=a_ssem.at[o - 1],
                recv_sem=a_rsem.at[3 - o],
                device_id=(lax.rem(my + o, N_DEV),),
                device_id_type=pl.DeviceIdType.MESH,
            )
            r.start()
            ardmas.append(r)
        for r in ardmas:
            r.wait()
        gmax = jnp.maximum(mloc, jnp.max(amax_rbuf[...]))
        scale = gmax / 127.0

        def q_i(i, carry):
            d = lax.rem(i, 2)
            h = i // 2
            g[d, :, pl.ds(h * HT, HT)] = (
                jnp.clip(
                    jnp.round(g[d, :, pl.ds(h * HT, HT)] / scale), 0.0, 127.0
                )
                * scale
            )
            return carry

        lax.fori_loop(0, 4, q_i, 0)
        stores = []
        for d in (0, 1):
            cp = pltpu.make_async_copy(
                g.at[d], o_ref.at[:, pl.ds(d * NH, NH)], csem.at[d]
            )
            cp.start()
            stores.append(cp)
        for cp in stores:
            cp.wait()

    outs = pl.pallas_call(
        body,
        out_shape=(
            jax.ShapeDtypeStruct((M_PER, N), F32),
            jax.ShapeDtypeStruct((2, 3, M_PER, NH), BF16),
            jax.ShapeDtypeStruct((2, M_PER, NH), BF16),
        ),
        in_specs=[
            pl.BlockSpec(memory_space=pl.ANY),
            pl.BlockSpec(memory_space=pl.ANY),
        ],
        out_specs=(
            pl.BlockSpec(memory_space=pl.ANY),
            pl.BlockSpec(memory_space=pl.ANY),
            pl.BlockSpec(memory_space=pl.ANY),
        ),
        scratch_shapes=[
            pltpu.VMEM((2, M_PER, NH), F32),
            pltpu.VMEM((M_PER, K_LOC), F32),
            pltpu.VMEM((K_LOC, HT), F32),
            pltpu.VMEM((M_PER, HT), BF16),
            pltpu.VMEM((8, 128), F32),
            pltpu.VMEM((3, 8, 128), F32),
            pltpu.SemaphoreType.DMA((2, 3, 2)),
            pltpu.SemaphoreType.DMA((2, 3, 2)),
            pltpu.SemaphoreType.DMA((3,)),
            pltpu.SemaphoreType.DMA((3,)),
            pltpu.SemaphoreType.DMA((2,)),
        ],
        compiler_params=pltpu.CompilerParams(
            collective_id=0, vmem_limit_bytes=58 * 1024 * 1024
        ),
    )(x, w_mat)
    return outs[0]
